# baseline (speedup 1.0000x reference)
import functools

import jax
import jax.numpy as jnp
from jax import lax
from jax.experimental import pallas as pl
from jax.experimental.pallas import tpu as pltpu
from jax.experimental.pallas import tpu_sc as plsc

NFINE = 10000
EFINE = 160000
NCOARSE = 2000
ECOARSE = 32000
CH0 = 128
CH1 = 256

NUM_CORES = 2
NUM_SUBCORES = 16
NUM_WORKERS = NUM_CORES * NUM_SUBCORES
CHUNK = 64

_MESH = dict(core_axis_name="c", subcore_axis_name="s")


def _sc_segment_sum(n_pad, e_pad, rows_per_tile, halves, nbuf):
    per_tile = e_pad // NUM_WORKERS
    n_chunks = per_tile // CHUNK
    assert per_tile % CHUNK == 0 and n_pad == NUM_SUBCORES * rows_per_tile

    @functools.partial(
        pl.kernel,
        mesh=plsc.VectorSubcoreMesh(**_MESH),
        out_type=jax.ShapeDtypeStruct((NUM_CORES, halves, n_pad, CH0),
                                      jnp.float32),
        scratch_types=[
            pltpu.VMEM((n_chunks, CHUNK), jnp.int32),
            pltpu.VMEM((n_chunks, CHUNK), jnp.int32),
            pltpu.VMEM((nbuf, CHUNK, CH0), jnp.float32),
            pltpu.VMEM_SHARED((halves, n_pad, CH0), jnp.float32),
            pltpu.SemaphoreType.DMA((nbuf,)),
            pltpu.SemaphoreType.DMA((nbuf,)),
            pltpu.SemaphoreType.DMA,
        ],
    )
    def seg(src_hbm, dst_hbm, feat_hbm, zeros_hbm, out_hbm,
            src_i, dst_i, rows, acc, gsem, ssem, isem):
        ci = lax.axis_index("c")
        sid = lax.axis_index("s")
        wid = ci * NUM_SUBCORES + sid
        row0 = sid * rows_per_tile
        cp_s = pltpu.make_async_copy(src_hbm.at[wid], src_i, isem)
        cp_s.start()
        cp_d = pltpu.make_async_copy(dst_hbm.at[wid], dst_i, isem)
        cp_d.start()
        for h in range(halves):
            pltpu.sync_copy(zeros_hbm, acc.at[h, pl.ds(row0, rows_per_tile)])
        cp_s.wait()
        cp_d.wait()
        plsc.subcore_barrier()

        for h in range(halves):
            pltpu.async_copy(feat_hbm.at[h].at[src_i.at[0]],
                             rows.at[0], gsem.at[0])

            def ring(j, carry, h=h):
                b = j % nbuf
                bn = (j + 1) % nbuf
                pltpu.make_async_copy(feat_hbm.at[h].at[src_i.at[j]],
                                      rows.at[b], gsem.at[b]).wait()

                @pl.when(j + 1 < n_chunks)
                def _():
                    @pl.when(j + 1 >= nbuf)
                    def _():
                        pltpu.make_async_copy(
                            rows.at[bn], acc.at[h].at[dst_i.at[0]],
                            ssem.at[bn]).wait()
                    pltpu.async_copy(feat_hbm.at[h].at[src_i.at[j + 1]],
                                     rows.at[bn], gsem.at[bn])

                pltpu.async_copy(rows.at[b], acc.at[h].at[dst_i.at[j]],
                                 ssem.at[b], add=True)
                return carry

            lax.fori_loop(0, n_chunks, ring, 0)
            for k in range(nbuf):
                j = n_chunks - nbuf + k
                pltpu.make_async_copy(rows.at[j % nbuf],
                                      acc.at[h].at[dst_i.at[0]],
                                      ssem.at[j % nbuf]).wait()
        plsc.subcore_barrier()
        for h in range(halves):
            pltpu.sync_copy(acc.at[h, pl.ds(row0, rows_per_tile)],
                            out_hbm.at[ci, h, pl.ds(row0, rows_per_tile)])

    return seg


def _sc_gather(n_out_pad, c):
    per_tile = n_out_pad // NUM_WORKERS
    n_chunks = per_tile // CHUNK
    assert per_tile % CHUNK == 0

    nbuf = n_chunks

    @functools.partial(
        pl.kernel,
        mesh=plsc.VectorSubcoreMesh(**_MESH),
        out_type=jax.ShapeDtypeStruct((n_out_pad, c), jnp.float32),
        scratch_types=[
            pltpu.VMEM((n_chunks, CHUNK), jnp.int32),
            pltpu.VMEM((nbuf, CHUNK, c), jnp.float32),
            pltpu.SemaphoreType.DMA((nbuf,)),
            pltpu.SemaphoreType.DMA((nbuf,)),
            pltpu.SemaphoreType.DMA,
        ],
    )
    def gat(idx_hbm, table_hbm, out_hbm, idx_i, rows, gsem, wsem, isem):
        ci = lax.axis_index("c")
        sid = lax.axis_index("s")
        wid = ci * NUM_SUBCORES + sid
        base = wid * per_tile
        pltpu.sync_copy(idx_hbm.at[wid], idx_i)
        gathers = []
        for b in range(nbuf):
            gathers.append(pltpu.async_copy(
                table_hbm.at[idx_i.at[b]], rows.at[b], gsem.at[b]))
        writes = []
        for b in range(nbuf):
            gathers[b].wait()
            writes.append(pltpu.async_copy(
                rows.at[b], out_hbm.at[pl.ds(base + b * CHUNK, CHUNK)],
                wsem.at[b]))
        for b in range(nbuf):
            writes[b].wait()

    return gat


def _tc_matmul_stats(terms, n_rows, blk, stats=True):
    nblk = n_rows // blk
    assert nblk * blk == n_rows
    c = terms[0][1].shape[-1]
    args, in_specs = [], []
    for x, w, part in terms:
        if part is None:
            in_specs.append(pl.BlockSpec((blk, x.shape[-1]), lambda i: (i, 0)))
        else:
            nl = len(part)
            in_specs.append(pl.BlockSpec(
                (1,) * nl + (blk, x.shape[-1]),
                lambda i, t=part: t + (i, 0)))
        args.append(x)
    for x, w, part in terms:
        in_specs.append(pl.BlockSpec(w.shape, lambda i: (0,) * w.ndim))
        args.append(w)
    nt = len(terms)
    out_specs = [pl.BlockSpec((blk, c), lambda i: (i, 0))]
    out_shape = [jax.ShapeDtypeStruct((n_rows, c), jnp.float32)]
    if stats:
        out_specs.append(pl.BlockSpec((8, c), lambda i: (0, 0)))
        out_shape.append(jax.ShapeDtypeStruct((8, c), jnp.float32))
    kdims = [w.shape[-2] for _, w, _ in terms]

    def body(*refs):
        out_ref = refs[2 * nt]
        acc = None
        for j in range(nt):
            xv = refs[j][...].reshape(blk, kdims[j])
            t = lax.dot_general(xv, refs[nt + j][...], (((1,), (0,)), ((), ())),
                                precision=lax.Precision.HIGHEST,
                                preferred_element_type=jnp.float32)
            acc = t if acc is None else acc + t
        out_ref[...] = acc
        if stats:
            s_ref = refs[2 * nt + 1]

            @pl.when(pl.program_id(0) == 0)
            def _():
                s_ref[...] = jnp.zeros_like(s_ref)

            s_ref[0:1, :] += jnp.sum(acc, axis=0, keepdims=True)
            s_ref[1:2, :] += jnp.sum(acc * acc, axis=0, keepdims=True)

    res = pl.pallas_call(
        body, grid=(nblk,), in_specs=in_specs, out_specs=out_specs,
        out_shape=out_shape)(*args)
    return res if stats else res[0]


def _tc_stats(t, n_rows, blk):
    nblk = n_rows // blk
    c = t.shape[-1]

    def body(t_ref, s_ref):
        @pl.when(pl.program_id(0) == 0)
        def _():
            s_ref[...] = jnp.zeros_like(s_ref)

        x = t_ref[...]
        s_ref[0:1, :] += jnp.sum(x, axis=0, keepdims=True)
        s_ref[1:2, :] += jnp.sum(x * x, axis=0, keepdims=True)

    return pl.pallas_call(
        body, grid=(nblk,),
        in_specs=[pl.BlockSpec((blk, c), lambda i: (i, 0))],
        out_specs=pl.BlockSpec((8, c), lambda i: (0, 0)),
        out_shape=jax.ShapeDtypeStruct((8, c), jnp.float32))(t)


def _tc_bn_apply(t, stats_arr, gamma, beta, n_rows, blk, idn=None):
    nblk = n_rows // blk
    c = t.shape[-1]
    inv_n = 1.0 / n_rows
    args = [t, stats_arr, gamma.reshape(1, c), beta.reshape(1, c)]
    in_specs = [
        pl.BlockSpec((blk, c), lambda i: (i, 0)),
        pl.BlockSpec((8, c), lambda i: (0, 0)),
        pl.BlockSpec((1, c), lambda i: (0, 0)),
        pl.BlockSpec((1, c), lambda i: (0, 0)),
    ]
    if idn is not None:
        args.append(idn)
        in_specs.append(pl.BlockSpec((blk, c), lambda i: (i, 0)))
    has_idn = idn is not None

    def body(*refs):
        t_ref, s_ref, g_ref, b_ref = refs[:4]
        out_ref = refs[-1]
        mean = s_ref[0:1, :] * inv_n
        var = s_ref[1:2, :] * inv_n - mean * mean
        scale = g_ref[...] * lax.rsqrt(var + 1e-5)
        y = (t_ref[...] - mean) * scale + b_ref[...]
        if has_idn:
            y = y + refs[4][...]
        out_ref[...] = jnp.maximum(y, 0.0)

    return pl.pallas_call(
        body, grid=(nblk,), in_specs=in_specs,
        out_specs=pl.BlockSpec((blk, c), lambda i: (i, 0)),
        out_shape=jax.ShapeDtypeStruct((n_rows, c), jnp.float32))(*args)


def kernel(x, edge_index, cluster_ids, coarse_edge_index, enc_W, enc_gamma,
           enc_beta, W_down, down_gamma, down_beta, inner_W, inner_gamma,
           inner_beta, W_up, up_gamma, up_beta, dec0_Wc1, dec0_Wc2, dec0_proj,
           dec0_gamma, dec0_beta, dec1_W, dec1_gamma, dec1_beta):
    i32 = jnp.int32
    f32 = jnp.float32

    def padded_edges(src, dst, e_pad, junk):
        e = src.shape[0]
        src_s = jnp.concatenate([src, jnp.zeros((e_pad - e,), i32)])
        dst_s = jnp.concatenate([dst, jnp.full((e_pad - e,), junk, i32)])
        return (src_s.reshape(NUM_WORKERS, -1, CHUNK),
                dst_s.reshape(NUM_WORKERS, -1, CHUNK))

    e_pad = 163840
    src_f, dst_f = padded_edges(edge_index[0], edge_index[1], e_pad, NFINE)
    ec_pad = 32768
    src_c, dst_c = padded_edges(coarse_edge_index[0], coarse_edge_index[1],
                                ec_pad, NCOARSE)
    p_pad = 12288
    src_p, dst_p = padded_edges(jnp.arange(NFINE, dtype=i32), cluster_ids,
                                p_pad, NCOARSE)
    gidx = jnp.concatenate(
        [cluster_ids, jnp.zeros((p_pad - NFINE,), i32)]).reshape(NUM_WORKERS, -1, CHUNK)

    z_fine = jnp.zeros((640, CH0), f32)
    z_coarse = jnp.zeros((128, CH0), f32)

    seg_fine = _sc_segment_sum(10240, e_pad, 640, 1, 3)
    seg_coarse = _sc_segment_sum(2048, ec_pad, 128, 2, 4)
    seg_pool = _sc_segment_sum(2048, p_pad, 128, 1, 3)
    gather_up = _sc_gather(p_pad, CH0)

    def agg_fine(h, wnbr):
        a = seg_fine(src_f, dst_f, h.reshape(1, NFINE, CH0), z_fine)
        return [(a, wnbr, (0, 0)), (a, wnbr, (1, 0))]

    def agg_coarse(h, wnbr):
        hs = jnp.stack([h[:, :CH0], h[:, CH0:]])
        a = seg_coarse(src_c, dst_c, hs, z_coarse)
        return [(a, wnbr[:CH0], (0, 0)), (a, wnbr[:CH0], (1, 0)),
                (a, wnbr[CH0:], (0, 1)), (a, wnbr[CH0:], (1, 1))]

    def resblock(h, agg, wc1, wc2, g, b, n, blk):
        t, st = _tc_matmul_stats([(h, wc1[0], None)] + agg(h, wc1[1]), n, blk)
        o1 = _tc_bn_apply(t, st, g[0], b[0], n, blk)
        t2, st2 = _tc_matmul_stats([(o1, wc2[0], None)] + agg(o1, wc2[1]),
                                   n, blk)
        return _tc_bn_apply(t2, st2, g[1], b[1], n, blk, idn=h)

    h = x
    for i in range(2):
        h = resblock(h, agg_fine, enc_W[i, 0], enc_W[i, 1],
                     enc_gamma[i], enc_beta[i], NFINE, 1000)
    shortcut = h

    p = seg_pool(src_p, dst_p, h.reshape(1, NFINE, CH0), z_coarse)
    t, st = _tc_matmul_stats([(p, W_down, (0, 0)), (p, W_down, (1, 0))],
                             NCOARSE, 400)
    hc = _tc_bn_apply(t, st, down_gamma, down_beta, NCOARSE, 400)

    for i in range(2):
        hc = resblock(hc, agg_coarse, inner_W[i, 0], inner_W[i, 1],
                      inner_gamma[i], inner_beta[i], NCOARSE, 400)

    table = _tc_matmul_stats([(hc, W_up, None)], NCOARSE, 400, stats=False)
    hu_g = gather_up(gidx, table)
    st = _tc_stats(hu_g, NFINE, 1000)
    hu = _tc_bn_apply(hu_g, st, up_gamma, up_beta, NFINE, 1000)

    ws, wn = dec0_Wc1[0], dec0_Wc1[1]
    idn = _tc_matmul_stats(
        [(hu, dec0_proj[:CH0], None), (shortcut, dec0_proj[CH0:], None)],
        NFINE, 1000, stats=False)
    z = _tc_matmul_stats(
        [(hu, wn[:CH0], None), (shortcut, wn[CH0:], None)],
        NFINE, 1000, stats=False)
    eye = jnp.eye(CH0, dtype=jnp.float32)
    t, st = _tc_matmul_stats(
        [(hu, ws[:CH0], None), (shortcut, ws[CH0:], None)]
        + agg_fine(z, eye),
        NFINE, 1000)
    o1 = _tc_bn_apply(t, st, dec0_gamma[0], dec0_beta[0], NFINE, 1000)
    t2, st2 = _tc_matmul_stats(
        [(o1, dec0_Wc2[0], None)] + agg_fine(o1, dec0_Wc2[1]), NFINE, 1000)
    h = _tc_bn_apply(t2, st2, dec0_gamma[1], dec0_beta[1], NFINE, 1000,
                     idn=idn)

    h = resblock(h, agg_fine, dec1_W[0], dec1_W[1],
                 dec1_gamma, dec1_beta, NFINE, 1000)
    return h


# speedup vs baseline: 1.0450x; 1.0450x over previous
import functools

import jax
import jax.numpy as jnp
from jax import lax
from jax.experimental import pallas as pl
from jax.experimental.pallas import tpu as pltpu
from jax.experimental.pallas import tpu_sc as plsc

NFINE = 10000
EFINE = 160000
NCOARSE = 2000
ECOARSE = 32000
CH0 = 128
CH1 = 256

NUM_CORES = 2
NUM_SUBCORES = 16
NUM_WORKERS = NUM_CORES * NUM_SUBCORES
CHUNK = 128

_MESH = dict(core_axis_name="c", subcore_axis_name="s")


def _sc_segment_sum(n_pad, e_pad, rows_per_tile, halves, nbuf):
    per_tile = e_pad // NUM_WORKERS
    n_chunks = per_tile // CHUNK
    assert per_tile % CHUNK == 0 and n_pad == NUM_SUBCORES * rows_per_tile

    @functools.partial(
        pl.kernel,
        mesh=plsc.VectorSubcoreMesh(**_MESH),
        out_type=jax.ShapeDtypeStruct((NUM_CORES, halves, n_pad, CH0),
                                      jnp.float32),
        scratch_types=[
            pltpu.VMEM((n_chunks, CHUNK), jnp.int32),
            pltpu.VMEM((n_chunks, CHUNK), jnp.int32),
            pltpu.VMEM((nbuf, CHUNK, CH0), jnp.float32),
            pltpu.VMEM_SHARED((halves, n_pad, CH0), jnp.float32),
            pltpu.SemaphoreType.DMA((nbuf,)),
            pltpu.SemaphoreType.DMA((nbuf,)),
            pltpu.SemaphoreType.DMA,
        ],
    )
    def seg(src_hbm, dst_hbm, feat_hbm, zeros_hbm, out_hbm,
            src_i, dst_i, rows, acc, gsem, ssem, isem):
        ci = lax.axis_index("c")
        sid = lax.axis_index("s")
        wid = ci * NUM_SUBCORES + sid
        row0 = sid * rows_per_tile
        cp_s = pltpu.make_async_copy(src_hbm.at[wid], src_i, isem)
        cp_s.start()
        cp_d = pltpu.make_async_copy(dst_hbm.at[wid], dst_i, isem)
        cp_d.start()
        for h in range(halves):
            pltpu.sync_copy(zeros_hbm, acc.at[h, pl.ds(row0, rows_per_tile)])
        cp_s.wait()
        cp_d.wait()
        plsc.subcore_barrier()

        for h in range(halves):
            pltpu.async_copy(feat_hbm.at[h].at[src_i.at[0]],
                             rows.at[0], gsem.at[0])

            def ring(j, carry, h=h):
                b = j % nbuf
                bn = (j + 1) % nbuf
                pltpu.make_async_copy(feat_hbm.at[h].at[src_i.at[j]],
                                      rows.at[b], gsem.at[b]).wait()

                @pl.when(j + 1 < n_chunks)
                def _():
                    @pl.when(j + 1 >= nbuf)
                    def _():
                        pltpu.make_async_copy(
                            rows.at[bn], acc.at[h].at[dst_i.at[0]],
                            ssem.at[bn]).wait()
                    pltpu.async_copy(feat_hbm.at[h].at[src_i.at[j + 1]],
                                     rows.at[bn], gsem.at[bn])

                pltpu.async_copy(rows.at[b], acc.at[h].at[dst_i.at[j]],
                                 ssem.at[b], add=True)
                return carry

            lax.fori_loop(0, n_chunks, ring, 0)
            for k in range(nbuf):
                j = n_chunks - nbuf + k
                pltpu.make_async_copy(rows.at[j % nbuf],
                                      acc.at[h].at[dst_i.at[0]],
                                      ssem.at[j % nbuf]).wait()
        plsc.subcore_barrier()
        for h in range(halves):
            pltpu.sync_copy(acc.at[h, pl.ds(row0, rows_per_tile)],
                            out_hbm.at[ci, h, pl.ds(row0, rows_per_tile)])

    return seg


def _sc_gather(n_out_pad, c):
    per_tile = n_out_pad // NUM_WORKERS
    n_chunks = per_tile // CHUNK
    assert per_tile % CHUNK == 0

    nbuf = n_chunks

    @functools.partial(
        pl.kernel,
        mesh=plsc.VectorSubcoreMesh(**_MESH),
        out_type=jax.ShapeDtypeStruct((n_out_pad, c), jnp.float32),
        scratch_types=[
            pltpu.VMEM((n_chunks, CHUNK), jnp.int32),
            pltpu.VMEM((nbuf, CHUNK, c), jnp.float32),
            pltpu.SemaphoreType.DMA((nbuf,)),
            pltpu.SemaphoreType.DMA((nbuf,)),
            pltpu.SemaphoreType.DMA,
        ],
    )
    def gat(idx_hbm, table_hbm, out_hbm, idx_i, rows, gsem, wsem, isem):
        ci = lax.axis_index("c")
        sid = lax.axis_index("s")
        wid = ci * NUM_SUBCORES + sid
        base = wid * per_tile
        pltpu.sync_copy(idx_hbm.at[wid], idx_i)
        gathers = []
        for b in range(nbuf):
            gathers.append(pltpu.async_copy(
                table_hbm.at[idx_i.at[b]], rows.at[b], gsem.at[b]))
        writes = []
        for b in range(nbuf):
            gathers[b].wait()
            writes.append(pltpu.async_copy(
                rows.at[b], out_hbm.at[pl.ds(base + b * CHUNK, CHUNK)],
                wsem.at[b]))
        for b in range(nbuf):
            writes[b].wait()

    return gat


def _tc_matmul_stats(terms, n_rows, blk, stats=True):
    nblk = n_rows // blk
    assert nblk * blk == n_rows
    c = terms[0][1].shape[-1]
    args, in_specs = [], []
    for x, w, part in terms:
        if part is None:
            in_specs.append(pl.BlockSpec((blk, x.shape[-1]), lambda i: (i, 0)))
        else:
            nl = len(part)
            in_specs.append(pl.BlockSpec(
                (1,) * nl + (blk, x.shape[-1]),
                lambda i, t=part: t + (i, 0)))
        args.append(x)
    for x, w, part in terms:
        in_specs.append(pl.BlockSpec(w.shape, lambda i: (0,) * w.ndim))
        args.append(w)
    nt = len(terms)
    out_specs = [pl.BlockSpec((blk, c), lambda i: (i, 0))]
    out_shape = [jax.ShapeDtypeStruct((n_rows, c), jnp.float32)]
    if stats:
        out_specs.append(pl.BlockSpec((8, c), lambda i: (0, 0)))
        out_shape.append(jax.ShapeDtypeStruct((8, c), jnp.float32))
    kdims = [w.shape[-2] for _, w, _ in terms]

    def body(*refs):
        out_ref = refs[2 * nt]
        acc = None
        for j in range(nt):
            xv = refs[j][...].reshape(blk, kdims[j])
            t = lax.dot_general(xv, refs[nt + j][...], (((1,), (0,)), ((), ())),
                                precision=lax.Precision.HIGHEST,
                                preferred_element_type=jnp.float32)
            acc = t if acc is None else acc + t
        out_ref[...] = acc
        if stats:
            s_ref = refs[2 * nt + 1]

            @pl.when(pl.program_id(0) == 0)
            def _():
                s_ref[...] = jnp.zeros_like(s_ref)

            s_ref[0:1, :] += jnp.sum(acc, axis=0, keepdims=True)
            s_ref[1:2, :] += jnp.sum(acc * acc, axis=0, keepdims=True)

    res = pl.pallas_call(
        body, grid=(nblk,), in_specs=in_specs, out_specs=out_specs,
        out_shape=out_shape)(*args)
    return res if stats else res[0]


def _tc_stats(t, n_rows, blk):
    nblk = n_rows // blk
    c = t.shape[-1]

    def body(t_ref, s_ref):
        @pl.when(pl.program_id(0) == 0)
        def _():
            s_ref[...] = jnp.zeros_like(s_ref)

        x = t_ref[...]
        s_ref[0:1, :] += jnp.sum(x, axis=0, keepdims=True)
        s_ref[1:2, :] += jnp.sum(x * x, axis=0, keepdims=True)

    return pl.pallas_call(
        body, grid=(nblk,),
        in_specs=[pl.BlockSpec((blk, c), lambda i: (i, 0))],
        out_specs=pl.BlockSpec((8, c), lambda i: (0, 0)),
        out_shape=jax.ShapeDtypeStruct((8, c), jnp.float32))(t)


def _tc_bn_apply(t, stats_arr, gamma, beta, n_rows, blk, idn=None):
    nblk = n_rows // blk
    c = t.shape[-1]
    inv_n = 1.0 / n_rows
    args = [t, stats_arr, gamma.reshape(1, c), beta.reshape(1, c)]
    in_specs = [
        pl.BlockSpec((blk, c), lambda i: (i, 0)),
        pl.BlockSpec((8, c), lambda i: (0, 0)),
        pl.BlockSpec((1, c), lambda i: (0, 0)),
        pl.BlockSpec((1, c), lambda i: (0, 0)),
    ]
    if idn is not None:
        args.append(idn)
        in_specs.append(pl.BlockSpec((blk, c), lambda i: (i, 0)))
    has_idn = idn is not None

    def body(*refs):
        t_ref, s_ref, g_ref, b_ref = refs[:4]
        out_ref = refs[-1]
        mean = s_ref[0:1, :] * inv_n
        var = s_ref[1:2, :] * inv_n - mean * mean
        scale = g_ref[...] * lax.rsqrt(var + 1e-5)
        y = (t_ref[...] - mean) * scale + b_ref[...]
        if has_idn:
            y = y + refs[4][...]
        out_ref[...] = jnp.maximum(y, 0.0)

    return pl.pallas_call(
        body, grid=(nblk,), in_specs=in_specs,
        out_specs=pl.BlockSpec((blk, c), lambda i: (i, 0)),
        out_shape=jax.ShapeDtypeStruct((n_rows, c), jnp.float32))(*args)


def kernel(x, edge_index, cluster_ids, coarse_edge_index, enc_W, enc_gamma,
           enc_beta, W_down, down_gamma, down_beta, inner_W, inner_gamma,
           inner_beta, W_up, up_gamma, up_beta, dec0_Wc1, dec0_Wc2, dec0_proj,
           dec0_gamma, dec0_beta, dec1_W, dec1_gamma, dec1_beta):
    i32 = jnp.int32
    f32 = jnp.float32

    def padded_edges(src, dst, e_pad, junk):
        e = src.shape[0]
        src_s = jnp.concatenate([src, jnp.zeros((e_pad - e,), i32)])
        dst_s = jnp.concatenate([dst, jnp.full((e_pad - e,), junk, i32)])
        return (src_s.reshape(NUM_WORKERS, -1, CHUNK),
                dst_s.reshape(NUM_WORKERS, -1, CHUNK))

    e_pad = 163840
    src_f, dst_f = padded_edges(edge_index[0], edge_index[1], e_pad, NFINE)
    ec_pad = 32768
    src_c, dst_c = padded_edges(coarse_edge_index[0], coarse_edge_index[1],
                                ec_pad, NCOARSE)
    p_pad = 12288
    src_p, dst_p = padded_edges(jnp.arange(NFINE, dtype=i32), cluster_ids,
                                p_pad, NCOARSE)
    gidx = jnp.concatenate(
        [cluster_ids, jnp.zeros((p_pad - NFINE,), i32)]).reshape(NUM_WORKERS, -1, CHUNK)

    z_fine = jnp.zeros((640, CH0), f32)
    z_coarse = jnp.zeros((128, CH0), f32)

    seg_fine = _sc_segment_sum(10240, e_pad, 640, 1, 2)
    seg_coarse = _sc_segment_sum(2048, ec_pad, 128, 2, 4)
    seg_pool = _sc_segment_sum(2048, p_pad, 128, 1, 3)
    gather_up = _sc_gather(p_pad, CH0)

    def agg_fine(h, wnbr):
        a = seg_fine(src_f, dst_f, h.reshape(1, NFINE, CH0), z_fine)
        return [(a, wnbr, (0, 0)), (a, wnbr, (1, 0))]

    def agg_coarse(h, wnbr):
        hs = jnp.stack([h[:, :CH0], h[:, CH0:]])
        a = seg_coarse(src_c, dst_c, hs, z_coarse)
        return [(a, wnbr[:CH0], (0, 0)), (a, wnbr[:CH0], (1, 0)),
                (a, wnbr[CH0:], (0, 1)), (a, wnbr[CH0:], (1, 1))]

    def resblock(h, agg, wc1, wc2, g, b, n, blk):
        t, st = _tc_matmul_stats([(h, wc1[0], None)] + agg(h, wc1[1]), n, blk)
        o1 = _tc_bn_apply(t, st, g[0], b[0], n, blk)
        t2, st2 = _tc_matmul_stats([(o1, wc2[0], None)] + agg(o1, wc2[1]),
                                   n, blk)
        return _tc_bn_apply(t2, st2, g[1], b[1], n, blk, idn=h)

    h = x
    for i in range(2):
        h = resblock(h, agg_fine, enc_W[i, 0], enc_W[i, 1],
                     enc_gamma[i], enc_beta[i], NFINE, 1000)
    shortcut = h

    p = seg_pool(src_p, dst_p, h.reshape(1, NFINE, CH0), z_coarse)
    t, st = _tc_matmul_stats([(p, W_down, (0, 0)), (p, W_down, (1, 0))],
                             NCOARSE, 400)
    hc = _tc_bn_apply(t, st, down_gamma, down_beta, NCOARSE, 400)

    for i in range(2):
        hc = resblock(hc, agg_coarse, inner_W[i, 0], inner_W[i, 1],
                      inner_gamma[i], inner_beta[i], NCOARSE, 400)

    table = _tc_matmul_stats([(hc, W_up, None)], NCOARSE, 400, stats=False)
    hu_g = gather_up(gidx, table)
    st = _tc_stats(hu_g, NFINE, 1000)
    hu = _tc_bn_apply(hu_g, st, up_gamma, up_beta, NFINE, 1000)

    ws, wn = dec0_Wc1[0], dec0_Wc1[1]
    idn = _tc_matmul_stats(
        [(hu, dec0_proj[:CH0], None), (shortcut, dec0_proj[CH0:], None)],
        NFINE, 1000, stats=False)
    z = _tc_matmul_stats(
        [(hu, wn[:CH0], None), (shortcut, wn[CH0:], None)],
        NFINE, 1000, stats=False)
    eye = jnp.eye(CH0, dtype=jnp.float32)
    t, st = _tc_matmul_stats(
        [(hu, ws[:CH0], None), (shortcut, ws[CH0:], None)]
        + agg_fine(z, eye),
        NFINE, 1000)
    o1 = _tc_bn_apply(t, st, dec0_gamma[0], dec0_beta[0], NFINE, 1000)
    t2, st2 = _tc_matmul_stats(
        [(o1, dec0_Wc2[0], None)] + agg_fine(o1, dec0_Wc2[1]), NFINE, 1000)
    h = _tc_bn_apply(t2, st2, dec0_gamma[1], dec0_beta[1], NFINE, 1000,
                     idn=idn)

    h = resblock(h, agg_fine, dec1_W[0], dec1_W[1],
                 dec1_gamma, dec1_beta, NFINE, 1000)
    return h
